# baseline (speedup 1.0000x reference)
import functools

import jax
import jax.numpy as jnp
from jax import lax
from jax.experimental import pallas as pl
from jax.experimental.pallas import tpu as pltpu
from jax.experimental.pallas import tpu_sc as plsc

_BINS = 20
_LAST_TIME = 1.0
_BIN_WIDTH = _LAST_TIME / float(_BINS)
_N = 100000
_D = 16
_E = 100000

_NTAB = 102400
_NB = 4096
_GRID_I = _NTAB // _NB
_GROUPS = 3
_ROWS16 = _GROUPS * _NTAB * 8

_NW = 32
_S = 128
_NSUB = 25
_C = _S * _NSUB
_E_PAD = _NW * _C


def _tables_body(x0t_ref, vt_ref, qv_ref, acc):
    o = pl.program_id(1)

    @pl.when(o == 0)
    def _():
        acc[...] = x0t_ref[...]

    v8 = vt_ref[...]
    cur = acc[...]
    pieces = []
    for bi in range(8):
        pieces.append(cur)
        pieces.append(v8[bi])
        cur = cur + _BIN_WIDTH * v8[bi]
    qv_ref[0] = jnp.concatenate(pieces, axis=0).T
    acc[...] = cur


def _build_tables(x0, v):
    x0t = x0.T
    vt = jnp.transpose(v, (0, 2, 1))
    qv = pl.pallas_call(
        _tables_body,
        grid=(_GRID_I, _GROUPS),
        in_specs=[
            pl.BlockSpec((_D, _NB), lambda i, o: (0, i)),
            pl.BlockSpec((8, _D, _NB), lambda i, o: (o, 0, i)),
        ],
        out_specs=pl.BlockSpec((1, _NB, 256), lambda i, o: (o, i, 0)),
        out_shape=jax.ShapeDtypeStruct((_GROUPS, _NTAB, 256), jnp.float32),
        scratch_shapes=[pltpu.VMEM((_D, _NB), jnp.float32)],
    )(x0t, vt)
    return qv.reshape(_ROWS16, 2 * _D)


def _sc_event_body(qvtab, beta_h, fi_h, fj_h, ii_h, jj_h, rr_h, out_h,
                   fi_v, fj_v, ii_v, jj_v, rr_v, out_v,
                   ti, tj, bi, bj, sem):
    cid = lax.axis_index("c")
    sid = lax.axis_index("s")
    wid = sid * 2 + cid
    pltpu.sync_copy(fi_h.at[wid], fi_v)
    pltpu.sync_copy(fj_h.at[wid], fj_v)
    pltpu.sync_copy(ii_h.at[wid], ii_v)
    pltpu.sync_copy(jj_h.at[wid], jj_v)
    pltpu.sync_copy(rr_h.at[wid], rr_v)

    rows0 = lax.iota(jnp.int32, 16)

    def step(k, carry):
        c0 = pltpu.async_copy(qvtab.at[fi_v.at[k]], ti, sem)
        c1 = pltpu.async_copy(qvtab.at[fj_v.at[k]], tj, sem)
        c4 = pltpu.async_copy(beta_h.at[ii_v.at[k]], bi, sem)
        c5 = pltpu.async_copy(beta_h.at[jj_v.at[k]], bj, sem)
        c0.wait(); c1.wait(); c4.wait(); c5.wait()
        for g in range(_S // 16):
            rows = rows0 + (g * 16)
            rr_vec = rr_v[pl.ds(k * _S + g * 16, 16)]
            acc = bi[pl.ds(g * 16, 16)] + bj[pl.ds(g * 16, 16)]
            for d in range(_D):
                cols = jnp.full((16,), d, jnp.int32)
                cols_v = jnp.full((16,), _D + d, jnp.int32)
                q_i = plsc.load_gather(ti, [rows, cols])
                q_j = plsc.load_gather(tj, [rows, cols])
                v_i = plsc.load_gather(ti, [rows, cols_v])
                v_j = plsc.load_gather(tj, [rows, cols_v])
                x = (q_i - q_j) + rr_vec * (v_i - v_j)
                acc = acc - x * x
            out_v[pl.ds(k * _S + g * 16, 16)] = acc
        return carry

    lax.fori_loop(0, _NSUB, step, 0)
    pltpu.sync_copy(out_v, out_h.at[wid])


_SC_KERNEL_CACHE = []


def _sc_event_kernel(*args):
    if not _SC_KERNEL_CACHE:
        _SC_KERNEL_CACHE.append(_make_sc_event_kernel())
    return _SC_KERNEL_CACHE[0](*args)


def _make_sc_event_kernel():
    return functools.partial(
        pl.kernel,
        out_type=jax.ShapeDtypeStruct((_NW, _C), jnp.float32),
        mesh=plsc.VectorSubcoreMesh(core_axis_name="c", subcore_axis_name="s"),
        compiler_params=pltpu.CompilerParams(
            needs_layout_passes=False, use_tc_tiling_on_sc=False
        ),
        scratch_types=[
            pltpu.VMEM((_NSUB, _S), jnp.int32),
            pltpu.VMEM((_NSUB, _S), jnp.int32),
            pltpu.VMEM((_NSUB, _S), jnp.int32),
            pltpu.VMEM((_NSUB, _S), jnp.int32),
            pltpu.VMEM((_C,), jnp.float32),
            pltpu.VMEM((_C,), jnp.float32),
            pltpu.VMEM((_S, 2 * _D), jnp.float32),
            pltpu.VMEM((_S, 2 * _D), jnp.float32),
            pltpu.VMEM((_S,), jnp.float32),
            pltpu.VMEM((_S,), jnp.float32),
            pltpu.SemaphoreType.DMA,
        ],
    )(_sc_event_body)


def kernel(x0, v, beta, times_list, node_pairs):
    bin_idx = jnp.floor(times_list / _BIN_WIDTH).astype(jnp.int32)
    bin_idx = jnp.where(bin_idx == _BINS, _BINS - 1, bin_idx)
    bin_idx = jnp.clip(bin_idx, 0, _BINS - 1)
    residual = jnp.mod(times_list, _BIN_WIDTH)
    i_idx = node_pairs[0]
    j_idx = node_pairs[1]

    fi = (bin_idx // 8) * (_NTAB * 8) + i_idx * 8 + (bin_idx % 8)
    fj = (bin_idx // 8) * (_NTAB * 8) + j_idx * 8 + (bin_idx % 8)
    pad = _E_PAD - _E
    fi_p = jnp.pad(fi, (0, pad)).reshape(_NW, _NSUB, _S)
    fj_p = jnp.pad(fj, (0, pad)).reshape(_NW, _NSUB, _S)
    ii_p = jnp.pad(i_idx, (0, pad)).reshape(_NW, _NSUB, _S)
    jj_p = jnp.pad(j_idx, (0, pad)).reshape(_NW, _NSUB, _S)
    rr_p = jnp.pad(residual, (0, pad)).reshape(_NW, _C)

    qvtab = _build_tables(x0, v)

    out = _sc_event_kernel(qvtab, beta, fi_p, fj_p, ii_p, jj_p, rr_p)
    return out.reshape(_E_PAD)[:_E]


# speedup vs baseline: 1.3472x; 1.3472x over previous
import functools

import jax
import jax.numpy as jnp
from jax import lax
from jax.experimental import pallas as pl
from jax.experimental.pallas import tpu as pltpu
from jax.experimental.pallas import tpu_sc as plsc

_BINS = 20
_LAST_TIME = 1.0
_BIN_WIDTH = _LAST_TIME / float(_BINS)
_N = 100000
_D = 16
_E = 100000

_NTAB = 102400
_NB = 4096
_GRID_I = _NTAB // _NB
_GROUPS = 3
_ROWS16 = _GROUPS * _NTAB * 8

_NW = 32
_S = 128
_NSUB = 25
_C = _S * _NSUB
_E_PAD = _NW * _C


def _tables_body(x0t_ref, vt_ref, qv_ref, acc):
    o = pl.program_id(1)
    h = pl.program_id(2)

    @pl.when((o == 0) & (h == 0))
    def _():
        acc[...] = x0t_ref[...]

    v4 = vt_ref[...]
    cur = acc[...]
    pieces = []
    for bi in range(4):
        pieces.append(cur)
        pieces.append(v4[bi])
        cur = cur + _BIN_WIDTH * v4[bi]
    qv_ref[0, 0] = jnp.concatenate(pieces, axis=0).T
    acc[...] = cur


def _build_tables(x0, v):
    x0t = x0.T
    vt = jnp.transpose(v, (0, 2, 1))
    qv = pl.pallas_call(
        _tables_body,
        grid=(_GRID_I, _GROUPS, 2),
        in_specs=[
            pl.BlockSpec((_D, _NB), lambda i, o, h: (0, i)),
            pl.BlockSpec((4, _D, _NB), lambda i, o, h: (2 * o + h, 0, i)),
        ],
        out_specs=pl.BlockSpec((1, 1, _NB, 128), lambda i, o, h: (o, h, i, 0)),
        out_shape=jax.ShapeDtypeStruct((_GROUPS, 2, _NTAB, 128), jnp.float32),
        scratch_shapes=[pltpu.VMEM((_D, _NB), jnp.float32)],
    )(x0t, vt)
    return qv.reshape(_ROWS16, 2 * _D)


def _sc_event_body(qvtab, beta_h, fi_h, fj_h, ii_h, jj_h, rr_h, out_h,
                   fi_v, fj_v, ii_v, jj_v, rr_v, out_v,
                   ti, tj, bi, bj, sem):
    cid = lax.axis_index("c")
    sid = lax.axis_index("s")
    wid = sid * 2 + cid
    pltpu.sync_copy(fi_h.at[wid], fi_v)
    pltpu.sync_copy(fj_h.at[wid], fj_v)
    pltpu.sync_copy(ii_h.at[wid], ii_v)
    pltpu.sync_copy(jj_h.at[wid], jj_v)
    pltpu.sync_copy(rr_h.at[wid], rr_v)

    rows0 = lax.iota(jnp.int32, 16)

    def step(k, carry):
        c0 = pltpu.async_copy(qvtab.at[fi_v.at[k]], ti, sem)
        c1 = pltpu.async_copy(qvtab.at[fj_v.at[k]], tj, sem)
        c4 = pltpu.async_copy(beta_h.at[ii_v.at[k]], bi, sem)
        c5 = pltpu.async_copy(beta_h.at[jj_v.at[k]], bj, sem)
        c0.wait(); c1.wait(); c4.wait(); c5.wait()
        for g in range(_S // 16):
            rows = rows0 + (g * 16)
            rr_vec = rr_v[pl.ds(k * _S + g * 16, 16)]
            acc = bi[pl.ds(g * 16, 16)] + bj[pl.ds(g * 16, 16)]
            for d in range(_D):
                cols = jnp.full((16,), d, jnp.int32)
                cols_v = jnp.full((16,), _D + d, jnp.int32)
                q_i = plsc.load_gather(ti, [rows, cols])
                q_j = plsc.load_gather(tj, [rows, cols])
                v_i = plsc.load_gather(ti, [rows, cols_v])
                v_j = plsc.load_gather(tj, [rows, cols_v])
                x = (q_i - q_j) + rr_vec * (v_i - v_j)
                acc = acc - x * x
            out_v[pl.ds(k * _S + g * 16, 16)] = acc
        return carry

    lax.fori_loop(0, _NSUB, step, 0)
    pltpu.sync_copy(out_v, out_h.at[wid])


_SC_KERNEL_CACHE = []


def _sc_event_kernel(*args):
    if not _SC_KERNEL_CACHE:
        _SC_KERNEL_CACHE.append(_make_sc_event_kernel())
    return _SC_KERNEL_CACHE[0](*args)


def _make_sc_event_kernel():
    return functools.partial(
        pl.kernel,
        out_type=jax.ShapeDtypeStruct((_NW, _C), jnp.float32),
        mesh=plsc.VectorSubcoreMesh(core_axis_name="c", subcore_axis_name="s"),
        compiler_params=pltpu.CompilerParams(
            needs_layout_passes=False, use_tc_tiling_on_sc=False
        ),
        scratch_types=[
            pltpu.VMEM((_NSUB, _S), jnp.int32),
            pltpu.VMEM((_NSUB, _S), jnp.int32),
            pltpu.VMEM((_NSUB, _S), jnp.int32),
            pltpu.VMEM((_NSUB, _S), jnp.int32),
            pltpu.VMEM((_C,), jnp.float32),
            pltpu.VMEM((_C,), jnp.float32),
            pltpu.VMEM((_S, 2 * _D), jnp.float32),
            pltpu.VMEM((_S, 2 * _D), jnp.float32),
            pltpu.VMEM((_S,), jnp.float32),
            pltpu.VMEM((_S,), jnp.float32),
            pltpu.SemaphoreType.DMA,
        ],
    )(_sc_event_body)


def kernel(x0, v, beta, times_list, node_pairs):
    bin_idx = jnp.floor(times_list / _BIN_WIDTH).astype(jnp.int32)
    bin_idx = jnp.where(bin_idx == _BINS, _BINS - 1, bin_idx)
    bin_idx = jnp.clip(bin_idx, 0, _BINS - 1)
    residual = jnp.mod(times_list, _BIN_WIDTH)
    i_idx = node_pairs[0]
    j_idx = node_pairs[1]

    fi = (bin_idx // 4) * (_NTAB * 4) + i_idx * 4 + (bin_idx % 4)
    fj = (bin_idx // 4) * (_NTAB * 4) + j_idx * 4 + (bin_idx % 4)
    pad = _E_PAD - _E
    fi_p = jnp.pad(fi, (0, pad)).reshape(_NW, _NSUB, _S)
    fj_p = jnp.pad(fj, (0, pad)).reshape(_NW, _NSUB, _S)
    ii_p = jnp.pad(i_idx, (0, pad)).reshape(_NW, _NSUB, _S)
    jj_p = jnp.pad(j_idx, (0, pad)).reshape(_NW, _NSUB, _S)
    rr_p = jnp.pad(residual, (0, pad)).reshape(_NW, _C)

    qvtab = _build_tables(x0, v)

    out = _sc_event_kernel(qvtab, beta, fi_p, fj_p, ii_p, jj_p, rr_p)
    return out.reshape(_E_PAD)[:_E]


# speedup vs baseline: 1.9649x; 1.4585x over previous
import functools

import jax
import jax.numpy as jnp
from jax import lax
from jax.experimental import pallas as pl
from jax.experimental.pallas import tpu as pltpu
from jax.experimental.pallas import tpu_sc as plsc

_BINS = 20
_LAST_TIME = 1.0
_BIN_WIDTH = _LAST_TIME / float(_BINS)
_N = 100000
_D = 16
_E = 100000

_NTAB = 102400
_NB = 4096
_GRID_I = _NTAB // _NB
_GROUPS = 3
_ROWS16 = _GROUPS * _NTAB * 8

_NW = 32
_S = 128
_NSUB = 25
_C = _S * _NSUB
_E_PAD = _NW * _C


def _tables_body(x0t_ref, vt_ref, q_ref, vr_ref, acc):
    o = pl.program_id(1)

    @pl.when(o == 0)
    def _():
        acc[...] = x0t_ref[...]

    v8 = vt_ref[...]
    vr_ref[0] = v8.reshape(8 * _D, _NB).T
    cur = acc[...]
    pieces = []
    for bi in range(8):
        pieces.append(cur)
        cur = cur + _BIN_WIDTH * v8[bi]
    q_ref[0] = jnp.concatenate(pieces, axis=0).T
    acc[...] = cur


def _build_tables(x0, v):
    x0t = x0.T
    vt = jnp.transpose(v, (0, 2, 1))
    q, vr = pl.pallas_call(
        _tables_body,
        grid=(_GRID_I, _GROUPS),
        in_specs=[
            pl.BlockSpec((_D, _NB), lambda i, o: (0, i)),
            pl.BlockSpec((8, _D, _NB), lambda i, o: (o, 0, i)),
        ],
        out_specs=[
            pl.BlockSpec((1, _NB, 128), lambda i, o: (o, i, 0)),
            pl.BlockSpec((1, _NB, 128), lambda i, o: (o, i, 0)),
        ],
        out_shape=[
            jax.ShapeDtypeStruct((_GROUPS, _NTAB, 128), jnp.float32),
            jax.ShapeDtypeStruct((_GROUPS, _NTAB, 128), jnp.float32),
        ],
        scratch_shapes=[pltpu.VMEM((_D, _NB), jnp.float32)],
    )(x0t, vt)
    return (q.reshape(_ROWS16, _D), vr.reshape(_ROWS16, _D))


def _sc_event_body(qtab, vtab, beta_h, fi_h, fj_h, ii_h, jj_h, rr_h, out_h,
                   fi_v, fj_v, ii_v, jj_v, rr_v, out_v,
                   qi, qj, vi, vj, bi, bj, sem):
    cid = lax.axis_index("c")
    sid = lax.axis_index("s")
    wid = sid * 2 + cid
    pltpu.sync_copy(fi_h.at[wid], fi_v)
    pltpu.sync_copy(fj_h.at[wid], fj_v)
    pltpu.sync_copy(ii_h.at[wid], ii_v)
    pltpu.sync_copy(jj_h.at[wid], jj_v)
    pltpu.sync_copy(rr_h.at[wid], rr_v)

    rows0 = lax.iota(jnp.int32, 16)

    def step(k, carry):
        c0 = pltpu.async_copy(qtab.at[fi_v.at[k]], qi, sem)
        c1 = pltpu.async_copy(qtab.at[fj_v.at[k]], qj, sem)
        c2 = pltpu.async_copy(vtab.at[fi_v.at[k]], vi, sem)
        c3 = pltpu.async_copy(vtab.at[fj_v.at[k]], vj, sem)
        c4 = pltpu.async_copy(beta_h.at[ii_v.at[k]], bi, sem)
        c5 = pltpu.async_copy(beta_h.at[jj_v.at[k]], bj, sem)
        c0.wait(); c1.wait(); c2.wait(); c3.wait(); c4.wait(); c5.wait()
        for g in range(_S // 16):
            rows = rows0 + (g * 16)
            rr_vec = rr_v[pl.ds(k * _S + g * 16, 16)]
            acc = bi[pl.ds(g * 16, 16)] + bj[pl.ds(g * 16, 16)]
            for d in range(_D):
                cols = jnp.full((16,), d, jnp.int32)
                q_i = plsc.load_gather(qi, [rows, cols])
                q_j = plsc.load_gather(qj, [rows, cols])
                v_i = plsc.load_gather(vi, [rows, cols])
                v_j = plsc.load_gather(vj, [rows, cols])
                x = (q_i - q_j) + rr_vec * (v_i - v_j)
                acc = acc - x * x
            out_v[pl.ds(k * _S + g * 16, 16)] = acc
        return carry

    lax.fori_loop(0, _NSUB, step, 0)
    pltpu.sync_copy(out_v, out_h.at[wid])


_SC_KERNEL_CACHE = []


def _sc_event_kernel(*args):
    if not _SC_KERNEL_CACHE:
        _SC_KERNEL_CACHE.append(_make_sc_event_kernel())
    return _SC_KERNEL_CACHE[0](*args)


def _make_sc_event_kernel():
    return functools.partial(
        pl.kernel,
        out_type=jax.ShapeDtypeStruct((_NW, _C), jnp.float32),
        mesh=plsc.VectorSubcoreMesh(core_axis_name="c", subcore_axis_name="s"),
        compiler_params=pltpu.CompilerParams(
            needs_layout_passes=False, use_tc_tiling_on_sc=False
        ),
        scratch_types=[
            pltpu.VMEM((_NSUB, _S), jnp.int32),
            pltpu.VMEM((_NSUB, _S), jnp.int32),
            pltpu.VMEM((_NSUB, _S), jnp.int32),
            pltpu.VMEM((_NSUB, _S), jnp.int32),
            pltpu.VMEM((_C,), jnp.float32),
            pltpu.VMEM((_C,), jnp.float32),
            pltpu.VMEM((_S, _D), jnp.float32),
            pltpu.VMEM((_S, _D), jnp.float32),
            pltpu.VMEM((_S, _D), jnp.float32),
            pltpu.VMEM((_S, _D), jnp.float32),
            pltpu.VMEM((_S,), jnp.float32),
            pltpu.VMEM((_S,), jnp.float32),
            pltpu.SemaphoreType.DMA,
        ],
    )(_sc_event_body)


def kernel(x0, v, beta, times_list, node_pairs):
    bin_idx = jnp.floor(times_list / _BIN_WIDTH).astype(jnp.int32)
    bin_idx = jnp.where(bin_idx == _BINS, _BINS - 1, bin_idx)
    bin_idx = jnp.clip(bin_idx, 0, _BINS - 1)
    residual = jnp.mod(times_list, _BIN_WIDTH)
    i_idx = node_pairs[0]
    j_idx = node_pairs[1]

    fi = (bin_idx // 8) * (_NTAB * 8) + i_idx * 8 + (bin_idx % 8)
    fj = (bin_idx // 8) * (_NTAB * 8) + j_idx * 8 + (bin_idx % 8)
    pad = _E_PAD - _E
    fi_p = jnp.pad(fi, (0, pad)).reshape(_NW, _NSUB, _S)
    fj_p = jnp.pad(fj, (0, pad)).reshape(_NW, _NSUB, _S)
    ii_p = jnp.pad(i_idx, (0, pad)).reshape(_NW, _NSUB, _S)
    jj_p = jnp.pad(j_idx, (0, pad)).reshape(_NW, _NSUB, _S)
    rr_p = jnp.pad(residual, (0, pad)).reshape(_NW, _C)

    qtab, vtab = _build_tables(x0, v)

    out = _sc_event_kernel(qtab, vtab, beta, fi_p, fj_p, ii_p, jj_p, rr_p)
    return out.reshape(_E_PAD)[:_E]


# speedup vs baseline: 2.3254x; 1.1835x over previous
import functools

import jax
import jax.numpy as jnp
from jax import lax
from jax.experimental import pallas as pl
from jax.experimental.pallas import tpu as pltpu
from jax.experimental.pallas import tpu_sc as plsc

_BINS = 20
_LAST_TIME = 1.0
_BIN_WIDTH = _LAST_TIME / float(_BINS)
_N = 100000
_D = 16
_E = 100000

_NTAB = 102400
_NB = 4096
_GRID_I = _NTAB // _NB
_GROUPS = 3
_ROWS16 = _GROUPS * _NTAB * 8

_NW = 32
_S = 128
_NSUB = 25
_C = _S * _NSUB
_E_PAD = _NW * _C


def _tables_body(x0t_ref, vt_ref, q_ref, vr_ref, acc):
    o = pl.program_id(1)

    @pl.when(o == 0)
    def _():
        acc[...] = x0t_ref[...]

    v8 = vt_ref[...]
    vr_ref[0] = v8.reshape(8 * _D, _NB).T
    cur = acc[...]
    pieces = []
    for bi in range(8):
        pieces.append(cur)
        cur = cur + _BIN_WIDTH * v8[bi]
    q_ref[0] = jnp.concatenate(pieces, axis=0).T
    acc[...] = cur


def _build_tables(x0, v):
    x0t = x0.T
    vt = jnp.transpose(v, (0, 2, 1))
    q, vr = pl.pallas_call(
        _tables_body,
        grid=(_GRID_I, _GROUPS),
        in_specs=[
            pl.BlockSpec((_D, _NB), lambda i, o: (0, i)),
            pl.BlockSpec((8, _D, _NB), lambda i, o: (o, 0, i)),
        ],
        out_specs=[
            pl.BlockSpec((1, _NB, 128), lambda i, o: (o, i, 0)),
            pl.BlockSpec((1, _NB, 128), lambda i, o: (o, i, 0)),
        ],
        out_shape=[
            jax.ShapeDtypeStruct((_GROUPS, _NTAB, 128), jnp.float32),
            jax.ShapeDtypeStruct((_GROUPS, _NTAB, 128), jnp.float32),
        ],
        scratch_shapes=[pltpu.VMEM((_D, _NB), jnp.float32)],
    )(x0t, vt)
    return (q.reshape(_ROWS16, _D), vr.reshape(_ROWS16, _D))


def _sc_event_body(qtab, vtab, beta_h, fi_h, fj_h, ii_h, jj_h, rr_h, out_h,
                   fi_v, fj_v, ii_v, jj_v, rr_v, out_v,
                   qi_a, qj_a, vi_a, vj_a, bi_a, bj_a,
                   qi_b, qj_b, vi_b, vj_b, bi_b, bj_b,
                   sem_a, sem_b):
    cid = lax.axis_index("c")
    sid = lax.axis_index("s")
    wid = sid * 2 + cid
    pltpu.sync_copy(fi_h.at[wid], fi_v)
    pltpu.sync_copy(fj_h.at[wid], fj_v)
    pltpu.sync_copy(ii_h.at[wid], ii_v)
    pltpu.sync_copy(jj_h.at[wid], jj_v)
    pltpu.sync_copy(rr_h.at[wid], rr_v)

    rows0 = lax.iota(jnp.int32, 16)
    set_a = (qi_a, qj_a, vi_a, vj_a, bi_a, bj_a, sem_a)
    set_b = (qi_b, qj_b, vi_b, vj_b, bi_b, bj_b, sem_b)

    def issue(k, bufs):
        qi, qj, vi, vj, bi, bj, sem = bufs
        return (pltpu.async_copy(qtab.at[fi_v.at[k]], qi, sem),
                pltpu.async_copy(qtab.at[fj_v.at[k]], qj, sem),
                pltpu.async_copy(vtab.at[fi_v.at[k]], vi, sem),
                pltpu.async_copy(vtab.at[fj_v.at[k]], vj, sem),
                pltpu.async_copy(beta_h.at[ii_v.at[k]], bi, sem),
                pltpu.async_copy(beta_h.at[jj_v.at[k]], bj, sem))

    def drain(k, bufs):
        qi, qj, vi, vj, bi, bj, sem = bufs
        for dst in (qi, qj, vi, vj, bi, bj):
            pltpu.make_async_copy(qtab.at[fi_v.at[k]] if dst.shape == qi.shape
                                  else beta_h.at[ii_v.at[k]], dst, sem).wait()

    def compute(k, bufs):
        qi, qj, vi, vj, bi, bj, _ = bufs
        for g in range(_S // 16):
            rows = rows0 + (g * 16)
            rr_vec = rr_v[pl.ds(k * _S + g * 16, 16)]
            acc = bi[pl.ds(g * 16, 16)] + bj[pl.ds(g * 16, 16)]
            for d in range(_D):
                cols = jnp.full((16,), d, jnp.int32)
                q_i = plsc.load_gather(qi, [rows, cols])
                q_j = plsc.load_gather(qj, [rows, cols])
                v_i = plsc.load_gather(vi, [rows, cols])
                v_j = plsc.load_gather(vj, [rows, cols])
                x = (q_i - q_j) + rr_vec * (v_i - v_j)
                acc = acc - x * x
            out_v[pl.ds(k * _S + g * 16, 16)] = acc

    issue(0, set_a)

    def step2(m, carry):
        k = m * 2
        issue(k + 1, set_b)
        drain(k, set_a)
        compute(k, set_a)
        issue(k + 2, set_a)
        drain(k + 1, set_b)
        compute(k + 1, set_b)
        return carry

    lax.fori_loop(0, (_NSUB - 1) // 2, step2, 0)
    drain(_NSUB - 1, set_a)
    compute(_NSUB - 1, set_a)
    pltpu.sync_copy(out_v, out_h.at[wid])


_SC_KERNEL_CACHE = []


def _sc_event_kernel(*args):
    if not _SC_KERNEL_CACHE:
        _SC_KERNEL_CACHE.append(_make_sc_event_kernel())
    return _SC_KERNEL_CACHE[0](*args)


def _make_sc_event_kernel():
    return functools.partial(
        pl.kernel,
        out_type=jax.ShapeDtypeStruct((_NW, _C), jnp.float32),
        mesh=plsc.VectorSubcoreMesh(core_axis_name="c", subcore_axis_name="s"),
        compiler_params=pltpu.CompilerParams(
            needs_layout_passes=False, use_tc_tiling_on_sc=False
        ),
        scratch_types=[
            pltpu.VMEM((_NSUB, _S), jnp.int32),
            pltpu.VMEM((_NSUB, _S), jnp.int32),
            pltpu.VMEM((_NSUB, _S), jnp.int32),
            pltpu.VMEM((_NSUB, _S), jnp.int32),
            pltpu.VMEM((_C,), jnp.float32),
            pltpu.VMEM((_C,), jnp.float32),
            pltpu.VMEM((_S, _D), jnp.float32),
            pltpu.VMEM((_S, _D), jnp.float32),
            pltpu.VMEM((_S, _D), jnp.float32),
            pltpu.VMEM((_S, _D), jnp.float32),
            pltpu.VMEM((_S,), jnp.float32),
            pltpu.VMEM((_S,), jnp.float32),
            pltpu.VMEM((_S, _D), jnp.float32),
            pltpu.VMEM((_S, _D), jnp.float32),
            pltpu.VMEM((_S, _D), jnp.float32),
            pltpu.VMEM((_S, _D), jnp.float32),
            pltpu.VMEM((_S,), jnp.float32),
            pltpu.VMEM((_S,), jnp.float32),
            pltpu.SemaphoreType.DMA,
            pltpu.SemaphoreType.DMA,
        ],
    )(_sc_event_body)


def kernel(x0, v, beta, times_list, node_pairs):
    bin_idx = jnp.floor(times_list / _BIN_WIDTH).astype(jnp.int32)
    bin_idx = jnp.where(bin_idx == _BINS, _BINS - 1, bin_idx)
    bin_idx = jnp.clip(bin_idx, 0, _BINS - 1)
    residual = jnp.mod(times_list, _BIN_WIDTH)
    i_idx = node_pairs[0]
    j_idx = node_pairs[1]

    fi = (bin_idx // 8) * (_NTAB * 8) + i_idx * 8 + (bin_idx % 8)
    fj = (bin_idx // 8) * (_NTAB * 8) + j_idx * 8 + (bin_idx % 8)
    pad = _E_PAD - _E
    fi_p = jnp.pad(fi, (0, pad)).reshape(_NW, _NSUB, _S)
    fj_p = jnp.pad(fj, (0, pad)).reshape(_NW, _NSUB, _S)
    ii_p = jnp.pad(i_idx, (0, pad)).reshape(_NW, _NSUB, _S)
    jj_p = jnp.pad(j_idx, (0, pad)).reshape(_NW, _NSUB, _S)
    rr_p = jnp.pad(residual, (0, pad)).reshape(_NW, _C)

    qtab, vtab = _build_tables(x0, v)

    out = _sc_event_kernel(qtab, vtab, beta, fi_p, fj_p, ii_p, jj_p, rr_p)
    return out.reshape(_E_PAD)[:_E]
